# Initial kernel scaffold; baseline (speedup 1.0000x reference)
import jax
import jax.numpy as jnp
from jax.experimental import pallas as pl


def kernel(hidden_states, prompt_lens):
    raise NotImplementedError("write your pallas kernel here")


# speedup vs baseline: 13.9102x; 13.9102x over previous
import jax
import jax.numpy as jnp
from jax.experimental import pallas as pl
from jax.experimental.pallas import tpu as pltpu


_BATCH = 16
_BLK = 512


def _pool_kernel(starts_ref, ends_ref, x_ref, out_ref):
    i = pl.program_id(0)
    nsteps = pl.num_programs(0)
    blk = x_ref.shape[0]
    r = i * blk + jax.lax.broadcasted_iota(jnp.int32, (1, blk), 1)
    rows = []
    for s in range(_BATCH):
        start_s = starts_ref[s]
        end_s = ends_ref[s]
        inv = 1.0 / (end_s - start_s).astype(jnp.float32)
        mask = (r >= start_s) & (r < end_s)
        rows.append(jnp.where(mask, inv, 0.0))
    w = jnp.concatenate(rows, axis=0)
    partial = jnp.dot(w, x_ref[...], preferred_element_type=jnp.float32)

    @pl.when(i == 0)
    def _init():
        out_ref[...] = partial

    @pl.when(i > 0)
    def _acc():
        out_ref[...] += partial


def kernel(hidden_states, prompt_lens):
    total, d = hidden_states.shape
    ends = jnp.cumsum(prompt_lens, dtype=jnp.int32)
    starts = ends - prompt_lens
    nsteps = total // _BLK

    grid_spec = pltpu.PrefetchScalarGridSpec(
        num_scalar_prefetch=2,
        grid=(nsteps,),
        in_specs=[
            pl.BlockSpec((_BLK, d), lambda i, starts, ends: (i, 0)),
        ],
        out_specs=pl.BlockSpec((_BATCH, d), lambda i, starts, ends: (0, 0)),
    )
    return pl.pallas_call(
        _pool_kernel,
        grid_spec=grid_spec,
        out_shape=jax.ShapeDtypeStruct((_BATCH, d), jnp.float32),
        compiler_params=pltpu.CompilerParams(
            dimension_semantics=("arbitrary",),
        ),
    )(starts, ends, hidden_states)
